# baseline (speedup 1.0000x reference)
import functools

import jax
import jax.numpy as jnp
from jax import lax
from jax.experimental import pallas as pl
from jax.experimental.pallas import tpu as pltpu
from jax.experimental.pallas import tpu_sc as plsc

B = 16384
F = 26
D = 16
FIELD_DIM = 100000
TOTAL_ROWS = F * FIELD_DIM

NUM_CORES = 2
NUM_SUBCORES = 16
NW = NUM_CORES * NUM_SUBCORES
ROWS_PER_W = B // NW
NB = 128
NCHUNK = ROWS_PER_W // NB


def _fm_body(xt_hbm, emb_hbm, lin_hbm, out_hbm, xv, idxv, rows, linv, outv,
             sem, lsem):
  wid = lax.axis_index("s") * NUM_CORES + lax.axis_index("c")
  base_w = wid * ROWS_PER_W

  def chunk_body(c, _):
    base = base_w + c * NB

    pltpu.sync_copy(xt_hbm.at[:, pl.ds(base, NB)], xv)

    def idx_body(j, _):
      sl = pl.ds(j * 16, 16)
      for f in range(F):
        idxv[f, sl] = xv[f, sl] + f * FIELD_DIM
      return 0

    lax.fori_loop(0, NB // 16, idx_body, 0)

    copies = []
    for f in range(F):
      copies.append(
          pltpu.async_copy(emb_hbm.at[idxv.at[f]], rows.at[f], sem))
      copies.append(
          pltpu.async_copy(lin_hbm.at[idxv.at[f]], linv.at[f], lsem))
    for cp in copies:
      cp.wait()

    def grp_body(j, _):
      sl = pl.ds(j * 16, 16)
      bvec = j * 16 + lax.iota(jnp.int32, 16)

      def d_body(d, out_j):
        dvec = jnp.full((16,), d, jnp.int32)
        acc = jnp.zeros((16,), jnp.float32)
        sq = jnp.zeros((16,), jnp.float32)
        for f in range(F):
          v = plsc.load_gather(rows, [jnp.full((16,), f, jnp.int32), bvec,
                                      dvec])
          acc = acc + v
          sq = sq + v * v
        return out_j + (acc * acc - sq)

      out_j = lax.fori_loop(0, D, d_body, jnp.zeros((16,), jnp.float32))

      lin_acc = linv[0, sl]
      for f in range(1, F):
        lin_acc = lin_acc + linv[f, sl]
      outv[sl] = lin_acc + 0.5 * out_j
      return 0

    lax.fori_loop(0, NB // 16, grp_body, 0)

    pltpu.sync_copy(outv, out_hbm.at[pl.ds(base, NB)])
    return 0

  lax.fori_loop(0, NCHUNK, chunk_body, 0)


@jax.jit
def _fm_sc(xt, emb_table, lin_flat):
  mesh = plsc.VectorSubcoreMesh(core_axis_name="c", subcore_axis_name="s")
  return pl.kernel(
      _fm_body,
      out_type=jax.ShapeDtypeStruct((B,), jnp.float32),
      mesh=mesh,
      compiler_params=pltpu.CompilerParams(needs_layout_passes=False,
                                           use_tc_tiling_on_sc=False),
      scratch_types=[
          pltpu.VMEM((F, NB), jnp.int32),
          pltpu.VMEM((F, NB), jnp.int32),
          pltpu.VMEM((F, NB, D), jnp.float32),
          pltpu.VMEM((F, NB), jnp.float32),
          pltpu.VMEM((NB,), jnp.float32),
          pltpu.SemaphoreType.DMA,
          pltpu.SemaphoreType.DMA,
      ],
  )(xt, emb_table, lin_flat)


def kernel(x, emb_table, lin_table):
  xt = jnp.asarray(x, jnp.int32).T
  lin_flat = lin_table.reshape(TOTAL_ROWS)
  out = _fm_sc(xt, emb_table, lin_flat)
  return out.reshape(B, 1)


# speedup vs baseline: 4.3821x; 4.3821x over previous
import functools

import jax
import jax.numpy as jnp
from jax import lax
from jax.experimental import pallas as pl
from jax.experimental.pallas import tpu as pltpu
from jax.experimental.pallas import tpu_sc as plsc

B = 16384
F = 26
D = 16
FIELD_DIM = 100000
TOTAL_ROWS = F * FIELD_DIM

LINE = 128
TBLK = 32768
NTBLK = (TOTAL_ROWS + TBLK - 1) // TBLK
NLINES = NTBLK * (TBLK // 8)

NUM_CORES = 2
NUM_SUBCORES = 16
NW = NUM_CORES * NUM_SUBCORES
ROWS_PER_W = B // NW
NB = 128
NCHUNK = ROWS_PER_W // NB


def _fm1_body(xt_hbm, emb_hbm, out_hbm, xv, gidx, rows, outv, sem):
  wid = lax.axis_index("s") * NUM_CORES + lax.axis_index("c")
  base_w = wid * ROWS_PER_W

  def chunk_body(c, _):
    base = base_w + c * NB
    pltpu.sync_copy(xt_hbm.at[:, pl.ds(base, NB)], xv)

    def idx_body(j, _):
      sl = pl.ds(j * 16, 16)
      for f in range(F):
        idx = xv[f, sl] + f * FIELD_DIM
        gidx[f, sl] = (
            lax.shift_left(lax.shift_right_logical(idx, 10), 10)
            + lax.shift_left(jnp.bitwise_and(idx, 127), 3)
            + jnp.bitwise_and(lax.shift_right_logical(idx, 7), 7))
      return 0

    lax.fori_loop(0, NB // 16, idx_body, 0)

    copies = [
        pltpu.async_copy(emb_hbm.at[gidx.at[f]], rows.at[f], sem)
        for f in range(F)
    ]
    for cp in copies:
      cp.wait()

    def grp_body(j, _):
      sl = pl.ds(j * 16, 16)
      bvec = j * 16 + lax.iota(jnp.int32, 16)

      def d_body(d, out_j):
        dvec = jnp.full((16,), d, jnp.int32)
        acc = jnp.zeros((16,), jnp.float32)
        sq = jnp.zeros((16,), jnp.float32)
        for f in range(F):
          v = plsc.load_gather(rows,
                               [jnp.full((16,), f, jnp.int32), bvec, dvec])
          acc = acc + v
          sq = sq + v * v
        return out_j + (acc * acc - sq)

      out_j = lax.fori_loop(0, D, d_body, jnp.zeros((16,), jnp.float32))
      outv[sl] = 0.5 * out_j
      return 0

    lax.fori_loop(0, NB // 16, grp_body, 0)

    pltpu.sync_copy(outv, out_hbm.at[pl.ds(base, NB)])
    return 0

  lax.fori_loop(0, NCHUNK, chunk_body, 0)


def _fm2_body(xt_hbm, lin_hbm, out1_hbm, out_hbm, xv, fidx, lgidx, linv, outv,
              lsem):
  wid = lax.axis_index("s") * NUM_CORES + lax.axis_index("c")
  base_w = wid * ROWS_PER_W

  def chunk_body(c, _):
    base = base_w + c * NB
    pltpu.sync_copy(xt_hbm.at[:, pl.ds(base, NB)], xv)

    def idx_body(j, _):
      sl = pl.ds(j * 16, 16)
      for f in range(F):
        idx = xv[f, sl] + f * FIELD_DIM
        fidx[f, sl] = idx
        lgidx[f, sl] = lax.shift_right_logical(idx, 3)
      return 0

    lax.fori_loop(0, NB // 16, idx_body, 0)

    copies = [
        pltpu.async_copy(lin_hbm.at[lgidx.at[f]], linv.at[f], lsem)
        for f in range(F)
    ]
    pltpu.sync_copy(out1_hbm.at[pl.ds(base, NB)], outv)
    for cp in copies:
      cp.wait()

    def grp_body(j, _):
      sl = pl.ds(j * 16, 16)
      bvec = j * 16 + lax.iota(jnp.int32, 16)
      lin_acc = jnp.zeros((16,), jnp.float32)
      for f in range(F):
        lane7 = jnp.bitwise_and(fidx[f, sl], 7)
        lin_acc = lin_acc + plsc.load_gather(
            linv, [jnp.full((16,), f, jnp.int32), bvec, lane7])
      outv[sl] = outv[sl] + lin_acc
      return 0

    lax.fori_loop(0, NB // 16, grp_body, 0)

    pltpu.sync_copy(outv, out_hbm.at[pl.ds(base, NB)])
    return 0

  lax.fori_loop(0, NCHUNK, chunk_body, 0)


def _sc_params():
  return pltpu.CompilerParams(needs_layout_passes=False,
                              use_tc_tiling_on_sc=False)


def _mesh():
  return plsc.VectorSubcoreMesh(core_axis_name="c", subcore_axis_name="s")


@jax.jit
def _fm_sc1(xt, emb_rows):
  return pl.kernel(
      _fm1_body,
      out_type=jax.ShapeDtypeStruct((B,), jnp.float32),
      mesh=_mesh(),
      compiler_params=_sc_params(),
      scratch_types=[
          pltpu.VMEM((F, NB), jnp.int32),
          pltpu.VMEM((F, NB), jnp.int32),
          pltpu.VMEM((F, NB, D), jnp.float32),
          pltpu.VMEM((NB,), jnp.float32),
          pltpu.SemaphoreType.DMA,
      ],
  )(xt, emb_rows)


@jax.jit
def _fm_sc2(xt, lin2d, out1):
  return pl.kernel(
      _fm2_body,
      out_type=jax.ShapeDtypeStruct((B,), jnp.float32),
      mesh=_mesh(),
      compiler_params=_sc_params(),
      scratch_types=[
          pltpu.VMEM((F, NB), jnp.int32),
          pltpu.VMEM((F, NB), jnp.int32),
          pltpu.VMEM((F, NB), jnp.int32),
          pltpu.VMEM((F, NB, 8), jnp.float32),
          pltpu.VMEM((NB,), jnp.float32),
          pltpu.SemaphoreType.DMA,
      ],
  )(xt, lin2d, out1)


def _tr_body(in_ref, out_ref):
  rowk = lax.broadcasted_iota(jnp.int32, (LINE, LINE), 0)
  colc = lax.broadcasted_iota(jnp.int32, (LINE, LINE), 1)
  perm = (rowk == 8 * (colc % D) + colc // D).astype(jnp.float32)
  for g in range(TBLK // 1024):
    a = in_ref[:, pl.ds(g * 1024, 1024)]
    in2 = a.reshape(LINE, LINE)
    out_ref[pl.ds(g * LINE, LINE), :] = lax.dot_general(
        in2, perm, (((0,), (0,)), ((), ())),
        preferred_element_type=jnp.float32)


@jax.jit
def _transpose_tc(emb_t):
  return pl.pallas_call(
      _tr_body,
      grid=(NTBLK,),
      in_specs=[pl.BlockSpec((D, TBLK), lambda i: (0, i))],
      out_specs=pl.BlockSpec((TBLK // 8, LINE), lambda i: (i, 0)),
      out_shape=jax.ShapeDtypeStruct((NLINES, LINE), jnp.float32),
  )(emb_t)


def kernel(x, emb_table, lin_table):
  xt = jnp.asarray(x, jnp.int32).T
  emb_rows = _transpose_tc(emb_table.T).reshape(NLINES * 8, D)
  out1 = _fm_sc1(xt, emb_rows)
  out = _fm_sc2(xt, lin_table.reshape(TOTAL_ROWS // 8, 8), out1)
  return out.reshape(B, 1)


# speedup vs baseline: 5.5102x; 1.2574x over previous
import functools

import jax
import jax.numpy as jnp
from jax import lax
from jax.experimental import pallas as pl
from jax.experimental.pallas import tpu as pltpu
from jax.experimental.pallas import tpu_sc as plsc

B = 16384
F = 26
D = 16
FIELD_DIM = 100000
TOTAL_ROWS = F * FIELD_DIM

LINE = 128
TBLK = 65536
NTBLK = (TOTAL_ROWS + TBLK - 1) // TBLK
NLINES = NTBLK * (TBLK // 8)

NUM_CORES = 2
NUM_SUBCORES = 16
NW = NUM_CORES * NUM_SUBCORES
ROWS_PER_W = B // NW
NB = 128
NCHUNK = ROWS_PER_W // NB


def _fm_body(xt_hbm, emb_hbm, lin_hbm, out_hbm, xv, gidx, fidx, lgidx, rows,
             linv, outv, sem, lsem):
  wid = lax.axis_index("s") * NUM_CORES + lax.axis_index("c")
  base_w = wid * ROWS_PER_W

  def chunk_body(c, _):
    base = base_w + c * NB

    pltpu.sync_copy(xt_hbm.at[:, pl.ds(base, NB)], xv)

    def idx_body(j, _):
      sl = pl.ds(j * 16, 16)
      for f in range(F):
        idx = xv[f, sl] + f * FIELD_DIM
        fidx[f, sl] = idx
        lgidx[f, sl] = lax.shift_right_logical(idx, 3)
        gidx[f, sl] = (
            lax.shift_left(lax.shift_right_logical(idx, 10), 10)
            + lax.shift_left(jnp.bitwise_and(idx, 127), 3)
            + jnp.bitwise_and(lax.shift_right_logical(idx, 7), 7))
      return 0

    lax.fori_loop(0, NB // 16, idx_body, 0)

    copies = []
    for f in range(F):
      copies.append(
          pltpu.async_copy(emb_hbm.at[gidx.at[f]], rows.at[f], sem))
      copies.append(
          pltpu.async_copy(lin_hbm.at[lgidx.at[f]], linv.at[f], lsem))
    for cp in copies:
      cp.wait()

    def grp_body(j, _):
      sl = pl.ds(j * 16, 16)
      bvec = j * 16 + lax.iota(jnp.int32, 16)

      def d_body(d, out_j):
        dvec = jnp.full((16,), d, jnp.int32)
        acc = jnp.zeros((16,), jnp.float32)
        sq = jnp.zeros((16,), jnp.float32)
        for f in range(F):
          v = plsc.load_gather(rows,
                               [jnp.full((16,), f, jnp.int32), bvec, dvec])
          acc = acc + v
          sq = sq + v * v
        return out_j + (acc * acc - sq)

      out_j = lax.fori_loop(0, D, d_body, jnp.zeros((16,), jnp.float32))

      lin_acc = jnp.zeros((16,), jnp.float32)
      for f in range(F):
        lane7 = jnp.bitwise_and(fidx[f, sl], 7)
        lin_acc = lin_acc + plsc.load_gather(
            linv, [jnp.full((16,), f, jnp.int32), bvec, lane7])
      outv[sl] = lin_acc + 0.5 * out_j
      return 0

    lax.fori_loop(0, NB // 16, grp_body, 0)

    pltpu.sync_copy(outv, out_hbm.at[pl.ds(base, NB)])
    return 0

  lax.fori_loop(0, NCHUNK, chunk_body, 0)


@jax.jit
def _fm_sc(xt, emb_rows, lin2d):
  mesh = plsc.VectorSubcoreMesh(core_axis_name="c", subcore_axis_name="s")
  return pl.kernel(
      _fm_body,
      out_type=jax.ShapeDtypeStruct((B,), jnp.float32),
      mesh=mesh,
      compiler_params=pltpu.CompilerParams(needs_layout_passes=False,
                                           use_tc_tiling_on_sc=False),
      scratch_types=[
          pltpu.VMEM((F, NB), jnp.int32),
          pltpu.VMEM((F, NB), jnp.int32),
          pltpu.VMEM((F, NB), jnp.int32),
          pltpu.VMEM((F, NB), jnp.int32),
          pltpu.VMEM((F, NB, D), jnp.float32),
          pltpu.VMEM((F, NB, 8), jnp.float32),
          pltpu.VMEM((NB,), jnp.float32),
          pltpu.SemaphoreType.DMA,
          pltpu.SemaphoreType.DMA,
      ],
  )(xt, emb_rows, lin2d)


def _tr_body(in_ref, lin_ref, out_ref, lout_ref):
  rowk = lax.broadcasted_iota(jnp.int32, (LINE, LINE), 0)
  colc = lax.broadcasted_iota(jnp.int32, (LINE, LINE), 1)
  perm = (rowk == 8 * (colc % D) + colc // D).astype(jnp.float32)
  for g in range(TBLK // 1024):
    a = in_ref[:, pl.ds(g * 1024, 1024)]
    in2 = a.reshape(LINE, LINE)
    out_ref[pl.ds(g * LINE, LINE), :] = lax.dot_general(
        in2, perm, (((0,), (0,)), ((), ())),
        preferred_element_type=jnp.float32)
  lout_ref[...] = lin_ref[0, :]


@jax.jit
def _transpose_tc(emb_t, lin_t):
  return pl.pallas_call(
      _tr_body,
      grid=(NTBLK,),
      in_specs=[
          pl.BlockSpec((D, TBLK), lambda i: (0, i)),
          pl.BlockSpec((1, TBLK), lambda i: (0, i)),
      ],
      out_specs=[
          pl.BlockSpec((TBLK // 8, LINE), lambda i: (i, 0)),
          pl.BlockSpec((TBLK,), lambda i: (i,)),
      ],
      out_shape=[
          jax.ShapeDtypeStruct((NLINES, LINE), jnp.float32),
          jax.ShapeDtypeStruct((NTBLK * TBLK,), jnp.float32),
      ],
  )(emb_t, lin_t)


def kernel(x, emb_table, lin_table):
  xt = jnp.asarray(x, jnp.int32).T
  emb_lines, lin_flat = _transpose_tc(emb_table.T, lin_table.T)
  emb_rows = emb_lines.reshape(NLINES * 8, D)
  out = _fm_sc(xt, emb_rows, lin_flat.reshape(NTBLK * TBLK // 8, 8))
  return out.reshape(B, 1)
